# baseline (speedup 1.0000x reference)
import functools

import jax
import jax.numpy as jnp
from jax import lax
from jax.experimental import pallas as pl
from jax.experimental.pallas import tpu as pltpu
from jax.experimental.pallas import tpu_sc as plsc

NC = 2
NS = 16
L = 16
NW = NC * NS

BATCH = 16384
D = 32
B_PER_W = BATCH // NW
N_GROUPS = B_PER_W // L


def _body(u_idx_hbm, i_idx_hbm, ut_hbm, st_hbm, it_hbm, out_hbm,
          uidx_v, iidx_v, pv, sv, iv, out_v, sem):
    wid = lax.axis_index("s") * NC + lax.axis_index("c")
    base = wid * B_PER_W

    pltpu.sync_copy(u_idx_hbm.at[pl.ds(base, B_PER_W)], uidx_v)
    pltpu.sync_copy(i_idx_hbm.at[pl.ds(base, B_PER_W)], iidx_v)

    copies = []
    for d in range(D):
        copies.append(pltpu.async_copy(ut_hbm.at[d].at[uidx_v], pv.at[d], sem))
        copies.append(pltpu.async_copy(st_hbm.at[d].at[uidx_v], sv.at[d], sem))
        copies.append(pltpu.async_copy(it_hbm.at[d].at[iidx_v], iv.at[d], sem))
    for c in copies:
        c.wait()

    def group(g, carry):
        b0 = g * L
        acc = jnp.zeros((L,), jnp.float32)
        for d in range(D):
            p = pv[d, pl.ds(b0, L)]
            s = sv[d, pl.ds(b0, L)]
            t = iv[d, pl.ds(b0, L)]
            acc = acc + (p + s) * t
        acc = acc * 0.5
        preds = 1.0 / (1.0 + jnp.exp(-acc))
        out_v[pl.ds(b0, L)] = preds
        return carry

    lax.fori_loop(0, N_GROUPS, group, 0)

    pltpu.sync_copy(out_v, out_hbm.at[pl.ds(base, B_PER_W)])


@jax.jit
def _run(u_idx, i_idx, ut, st, it):
    mesh = plsc.VectorSubcoreMesh(core_axis_name="c", subcore_axis_name="s")
    f = pl.kernel(
        _body,
        out_type=jax.ShapeDtypeStruct((BATCH,), jnp.float32),
        mesh=mesh,
        scratch_types=[
            pltpu.VMEM((B_PER_W,), jnp.int32),
            pltpu.VMEM((B_PER_W,), jnp.int32),
            pltpu.VMEM((D, B_PER_W), jnp.float32),
            pltpu.VMEM((D, B_PER_W), jnp.float32),
            pltpu.VMEM((D, B_PER_W), jnp.float32),
            pltpu.VMEM((B_PER_W,), jnp.float32),
            pltpu.SemaphoreType.DMA,
        ],
        compiler_params=pltpu.CompilerParams(needs_layout_passes=False,
                                             use_tc_tiling_on_sc=False),
    )
    return f(u_idx, i_idx, ut, st, it)


def kernel(user_inputs, u_item_inputs, user_table_private, user_table_shared,
           item_table):
    return _run(user_inputs, u_item_inputs, user_table_private.T,
                user_table_shared.T, item_table.T)


# speedup vs baseline: 5.8964x; 5.8964x over previous
import jax
import jax.numpy as jnp
from jax import lax
from jax.experimental import pallas as pl
from jax.experimental.pallas import tpu as pltpu
from jax.experimental.pallas import tpu_sc as plsc

NC = 2
NS = 16
L = 16
NW = NC * NS

BATCH = 16384
D = 32
B_PER_W = BATCH // NW
IDX_CHUNK = 128
N_CHUNKS = B_PER_W // IDX_CHUNK
N_GROUPS = B_PER_W // L


def _body(u_idx_hbm, i_idx_hbm, priv_hbm, shar_hbm, item_hbm, out_hbm,
          uidx_v, iidx_v, priv_v, shar_v, item_v, out_v, sem):
    wid = lax.axis_index("s") * NC + lax.axis_index("c")
    base = wid * B_PER_W

    row0 = wid * N_CHUNKS
    pltpu.sync_copy(u_idx_hbm.at[pl.ds(row0, N_CHUNKS)], uidx_v)
    pltpu.sync_copy(i_idx_hbm.at[pl.ds(row0, N_CHUNKS)], iidx_v)

    copies = []
    for j in range(N_CHUNKS):
        dst = pl.ds(j * IDX_CHUNK, IDX_CHUNK)
        copies.append(pltpu.async_copy(priv_hbm.at[uidx_v.at[j]],
                                       priv_v.at[dst], sem))
        copies.append(pltpu.async_copy(shar_hbm.at[uidx_v.at[j]],
                                       shar_v.at[dst], sem))
        copies.append(pltpu.async_copy(item_hbm.at[iidx_v.at[j]],
                                       item_v.at[dst], sem))
    for c in copies:
        c.wait()

    def group(g, carry):
        b0 = g * L
        rows = b0 + lax.iota(jnp.int32, L)
        acc = jnp.zeros((L,), jnp.float32)
        for j in range(D):
            col = jnp.full((L,), j, jnp.int32)
            p = plsc.load_gather(priv_v, [rows, col])
            s = plsc.load_gather(shar_v, [rows, col])
            it = plsc.load_gather(item_v, [rows, col])
            acc = acc + (p + s) * it
        acc = acc * 0.5
        preds = 1.0 / (1.0 + jnp.exp(-acc))
        out_v[pl.ds(b0, L)] = preds
        return carry

    lax.fori_loop(0, N_GROUPS, group, 0)

    pltpu.sync_copy(out_v, out_hbm.at[pl.ds(base, B_PER_W)])


@jax.jit
def _run(u_idx2d, i_idx2d, priv, shar, item):
    mesh = plsc.VectorSubcoreMesh(core_axis_name="c", subcore_axis_name="s")
    f = pl.kernel(
        _body,
        out_type=jax.ShapeDtypeStruct((BATCH,), jnp.float32),
        mesh=mesh,
        scratch_types=[
            pltpu.VMEM((N_CHUNKS, IDX_CHUNK), jnp.int32),
            pltpu.VMEM((N_CHUNKS, IDX_CHUNK), jnp.int32),
            pltpu.VMEM((B_PER_W, D), jnp.float32),
            pltpu.VMEM((B_PER_W, D), jnp.float32),
            pltpu.VMEM((B_PER_W, D), jnp.float32),
            pltpu.VMEM((B_PER_W,), jnp.float32),
            pltpu.SemaphoreType.DMA,
        ],
        compiler_params=pltpu.CompilerParams(needs_layout_passes=False,
                                             use_tc_tiling_on_sc=False),
    )
    return f(u_idx2d, i_idx2d, priv, shar, item)


def kernel(user_inputs, u_item_inputs, user_table_private, user_table_shared,
           item_table):
    u2d = user_inputs.reshape(BATCH // IDX_CHUNK, IDX_CHUNK)
    i2d = u_item_inputs.reshape(BATCH // IDX_CHUNK, IDX_CHUNK)
    return _run(u2d, i2d, user_table_private, user_table_shared, item_table)
